# baseline (speedup 1.0000x reference)
import functools

import jax
import jax.numpy as jnp
from jax import lax
from jax.experimental import pallas as pl
from jax.experimental.pallas import tpu as pltpu
from jax.experimental.pallas import tpu_sc as plsc

_B, _C, _A, _W, _O, _L, _K = 1024, 100, 6, 5, 4, 256, 2
_NC, _NS = 2, 16
_NW = _NC * _NS
_BPW = _B // _NW
_IPW = (_B * _K) // _NW
_NEG = float("-inf")


def _sc_stats(sel, num, agg, op, stab, etab, idx, colf, selt, numt, aggt,
              optf):
  mesh = plsc.VectorSubcoreMesh(core_axis_name="c", subcore_axis_name="s")
  f32 = jnp.float32

  @functools.partial(
      pl.kernel,
      mesh=mesh,
      out_type=[
          jax.ShapeDtypeStruct((_B * _K, _L), f32),
          jax.ShapeDtypeStruct((_B * _K, _L), f32),
          jax.ShapeDtypeStruct((_B,), f32),
          jax.ShapeDtypeStruct((_B,), f32),
          jax.ShapeDtypeStruct((_B,), f32),
          jax.ShapeDtypeStruct((_B,), f32),
          jax.ShapeDtypeStruct((_B,), f32),
          jax.ShapeDtypeStruct((_B,), f32),
          jax.ShapeDtypeStruct((_B * _K,), f32),
          jax.ShapeDtypeStruct((_B * _K,), f32),
      ],
      scratch_types=[
          pltpu.VMEM((_IPW,), jnp.int32),
          pltpu.VMEM((_IPW,), jnp.int32),
          pltpu.VMEM((_IPW,), jnp.int32),
          pltpu.VMEM((_BPW,), jnp.int32),
          pltpu.VMEM((_BPW,), jnp.int32),
          pltpu.VMEM((_BPW,), jnp.int32),
          pltpu.VMEM((_BPW * _C,), f32),
          pltpu.VMEM((_BPW * _W,), f32),
          pltpu.VMEM((_BPW * _C * _A,), f32),
          pltpu.VMEM((_BPW * _C * _O,), f32),
          pltpu.VMEM((_IPW, _L), f32),
          pltpu.VMEM((_IPW, _L), f32),
          pltpu.VMEM((_BPW,), f32),
          pltpu.VMEM((_BPW,), f32),
          pltpu.VMEM((_BPW,), f32),
          pltpu.VMEM((_BPW,), f32),
          pltpu.VMEM((_BPW,), f32),
          pltpu.VMEM((_BPW,), f32),
          pltpu.VMEM((_IPW,), f32),
          pltpu.VMEM((_IPW,), f32),
          pltpu.SemaphoreType.DMA,
          pltpu.SemaphoreType.DMA,
      ],
      compiler_params=pltpu.CompilerParams(needs_layout_passes=False),
  )
  def k(sel_hbm, num_hbm, agg_hbm, op_hbm, stab_hbm, etab_hbm, idx_hbm,
        colf_hbm, selt_hbm, numt_hbm, aggt_hbm, optf_hbm,
        srows_o, erows_o, sesel_o, mpsel_o, senum_o, mpnum_o, seagg_o,
        mpagg_o, seop_o, mpop_o,
        idx_v, colf_v, optf_v, selt_v, numt_v, aggt_v, sel_v, num_v, agg_v,
        op_v, sr_v, er_v, ses_v, mps_v, sen_v, mpn_v, sea_v, mpa_v, seo_v,
        mpo_v, sem_s, sem_e):
    wid = lax.axis_index("s") * _NC + lax.axis_index("c")
    b0 = wid * _BPW
    i0 = wid * _IPW
    iota = lax.iota(jnp.int32, 16)
    ones = jnp.full((16,), 1.0, f32)

    pltpu.sync_copy(idx_hbm.at[pl.ds(i0, _IPW)], idx_v)
    cp_s = pltpu.async_copy(stab_hbm.at[idx_v], sr_v, sem_s)
    cp_e = pltpu.async_copy(etab_hbm.at[idx_v], er_v, sem_e)

    pltpu.sync_copy(colf_hbm.at[pl.ds(i0, _IPW)], colf_v)
    pltpu.sync_copy(optf_hbm.at[pl.ds(i0, _IPW)], optf_v)
    pltpu.sync_copy(selt_hbm.at[pl.ds(b0, _BPW)], selt_v)
    pltpu.sync_copy(numt_hbm.at[pl.ds(b0, _BPW)], numt_v)
    pltpu.sync_copy(aggt_hbm.at[pl.ds(b0, _BPW)], aggt_v)
    pltpu.sync_copy(sel_hbm.at[pl.ds(b0 * _C, _BPW * _C)], sel_v)
    pltpu.sync_copy(num_hbm.at[pl.ds(b0 * _W, _BPW * _W)], num_v)
    pltpu.sync_copy(agg_hbm.at[pl.ds(b0 * _C * _A, _BPW * _C * _A)], agg_v)
    pltpu.sync_copy(op_hbm.at[pl.ds(b0 * _C * _O, _BPW * _C * _O)], op_v)

    def store1(ref, pos_splat, scalar):
      plsc.store_scatter(ref, [pos_splat], jnp.full((16,), scalar, f32),
                         mask=iota == 0)

    def b_body(b, carry):
      bs = jnp.full((16,), b, jnp.int32)
      selb = jnp.full((16,), b * _C, jnp.int32)
      m = jnp.full((16,), _NEG, f32)
      for j in range(7):
        cols = jnp.minimum(iota + (16 * j), _C - 1)
        x = plsc.load_gather(sel_v, [selb + cols])
        m = jnp.maximum(m, x)
      msel = jnp.max(m)
      msp = jnp.full((16,), msel, f32)
      se = jnp.zeros((16,), f32)
      amin = jnp.full((16,), 16384, jnp.int32)
      for j in range(7):
        raw = iota + (16 * j)
        cols = jnp.minimum(raw, _C - 1)
        valid = raw < _C
        x = plsc.load_gather(sel_v, [selb + cols])
        se = se + jnp.where(valid, jnp.exp(x - msp), 0.0)
        amin = jnp.minimum(amin, jnp.where(valid & (x == msp), cols, 16384))
      amax = jnp.min(amin)
      tsel = plsc.load_gather(selt_v, [bs])
      psel = jnp.max(plsc.load_gather(sel_v, [selb + tsel]))
      store1(ses_v, bs, jnp.sum(se))
      store1(mps_v, bs, msel - psel)
      numb = jnp.full((16,), b * _W, jnp.int32)
      xn = plsc.load_gather(num_v, [numb + jnp.minimum(iota, _W - 1)])
      vn = iota < _W
      mn = jnp.max(jnp.where(vn, xn, _NEG))
      sen = jnp.sum(jnp.where(vn, jnp.exp(xn - mn), 0.0))
      tn = plsc.load_gather(numt_v, [bs])
      pn = jnp.max(plsc.load_gather(num_v, [numb + tn]))
      store1(sen_v, bs, sen)
      store1(mpn_v, bs, mn - pn)
      aggb = jnp.full((16,), (b * _C + amax) * _A, jnp.int32)
      xa = plsc.load_gather(agg_v, [aggb + jnp.minimum(iota, _A - 1)])
      va = iota < _A
      ma = jnp.max(jnp.where(va, xa, _NEG))
      sea = jnp.sum(jnp.where(va, jnp.exp(xa - ma), 0.0))
      ta = plsc.load_gather(aggt_v, [bs])
      pa = jnp.max(plsc.load_gather(agg_v, [aggb + ta]))
      store1(sea_v, bs, sea)
      store1(mpa_v, bs, ma - pa)
      return carry

    lax.fori_loop(0, _BPW, b_body, 0)

    def i_body(i, carry):
      isp = jnp.full((16,), i, jnp.int32)
      colsp = plsc.load_gather(colf_v, [isp])
      opb = jnp.full((16,), (i // _K) * _C * _O, jnp.int32) + colsp * _O
      xo = plsc.load_gather(op_v, [opb + jnp.minimum(iota, _O - 1)])
      vo = iota < _O
      mo = jnp.max(jnp.where(vo, xo, _NEG))
      seo = jnp.sum(jnp.where(vo, jnp.exp(xo - mo), 0.0))
      to = plsc.load_gather(optf_v, [isp])
      po = jnp.max(plsc.load_gather(op_v, [opb + to]))
      store1(seo_v, isp, seo)
      store1(mpo_v, isp, mo - po)
      return carry

    lax.fori_loop(0, _IPW, i_body, 0)

    cp_s.wait()
    cp_e.wait()
    pltpu.sync_copy(sr_v, srows_o.at[pl.ds(i0, _IPW)])
    pltpu.sync_copy(er_v, erows_o.at[pl.ds(i0, _IPW)])
    pltpu.sync_copy(ses_v, sesel_o.at[pl.ds(b0, _BPW)])
    pltpu.sync_copy(mps_v, mpsel_o.at[pl.ds(b0, _BPW)])
    pltpu.sync_copy(sen_v, senum_o.at[pl.ds(b0, _BPW)])
    pltpu.sync_copy(mpn_v, mpnum_o.at[pl.ds(b0, _BPW)])
    pltpu.sync_copy(sea_v, seagg_o.at[pl.ds(b0, _BPW)])
    pltpu.sync_copy(mpa_v, mpagg_o.at[pl.ds(b0, _BPW)])
    pltpu.sync_copy(seo_v, seop_o.at[pl.ds(i0, _IPW)])
    pltpu.sync_copy(mpo_v, mpop_o.at[pl.ds(i0, _IPW)])

  return k(sel, num, agg, op, stab, etab, idx, colf, selt, numt, aggt, optf)


def _softplus(x):
  return jnp.maximum(x, 0.0) + jnp.log1p(jnp.exp(-jnp.abs(x)))


def _tc_loss_body(col_ref, srow_ref, erow_ref, colt_ref, stt_ref, ett_ref,
                  sesel_ref, mpsel_ref, senum_ref, mpnum_ref, seagg_ref,
                  mpagg_ref, seop_ref, mpop_ref, out_ref):
  loss = jnp.sum(mpsel_ref[...] + jnp.log(sesel_ref[...])) * (1.0 / _B)
  loss = loss + jnp.sum(mpagg_ref[...] + jnp.log(seagg_ref[...])) * (1.0 / _B)
  loss = loss + jnp.sum(mpnum_ref[...] + jnp.log(senum_ref[...])) * (1.0 / _B)
  loss = loss + jnp.sum(mpop_ref[...] + jnp.log(seop_ref[...])) * (
      1.0 / (_B * _K))

  colw = col_ref[...]
  cidx = lax.broadcasted_iota(jnp.int32, (_B, _C), 1)
  t0 = colt_ref[..., 0:1]
  t1 = colt_ref[..., 1:2]
  h = (cidx == t0) | (cidx == t1)
  sp_pos = _softplus(colw)
  sp_neg = sp_pos - colw
  bce_sum = jnp.sum(jnp.where(h, 3.0 * sp_neg, sp_pos))
  loss = loss + bce_sum * (jnp.float32(_B) / _C)

  jl = lax.broadcasted_iota(jnp.int32, (_B * _K, _L), 1)
  for rows_ref, tgt_ref in ((srow_ref, stt_ref), (erow_ref, ett_ref)):
    x = rows_ref[...]
    m = jnp.max(x, axis=1, keepdims=True)
    lse = m + jnp.log(jnp.sum(jnp.exp(x - m), axis=1, keepdims=True))
    picked = jnp.sum(jnp.where(jl == tgt_ref[...], x, 0.0), axis=1,
                     keepdims=True)
    loss = loss + jnp.sum(lse - picked) * (1.0 / (_B * _K))

  out_ref[...] = jnp.reshape(loss, (1, 1))


def kernel(agg_logits, sel_logits, where_num_logits, where_col_logits,
           where_op_logits, where_start_logits, where_end_logits,
           agg_target, sel_target, where_num_target, where_col_target,
           where_op_target, where_start_target, where_end_target):
  i32 = jnp.int32
  colt = where_col_target.astype(i32)
  colf = colt.reshape(-1)
  idx = (jnp.arange(_B, dtype=i32)[:, None] * _C + colt).reshape(-1)

  (srows, erows, se_sel, mp_sel, se_num, mp_num, se_agg, mp_agg, se_op,
   mp_op) = _sc_stats(
       sel_logits.reshape(-1), where_num_logits.reshape(-1),
       agg_logits.reshape(-1), where_op_logits.reshape(-1),
       where_start_logits.reshape(_B * _C, _L),
       where_end_logits.reshape(_B * _C, _L),
       idx, colf,
       sel_target.astype(i32), where_num_target.astype(i32),
       agg_target.astype(i32), where_op_target.astype(i32).reshape(-1))

  out = pl.pallas_call(
      _tc_loss_body,
      out_shape=jax.ShapeDtypeStruct((1, 1), jnp.float32),
  )(
      where_col_logits,
      srows,
      erows,
      colt,
      where_start_target.astype(i32).reshape(_B * _K, 1),
      where_end_target.astype(i32).reshape(_B * _K, 1),
      se_sel.reshape(_B // 128, 128),
      mp_sel.reshape(_B // 128, 128),
      se_num.reshape(_B // 128, 128),
      mp_num.reshape(_B // 128, 128),
      se_agg.reshape(_B // 128, 128),
      mp_agg.reshape(_B // 128, 128),
      se_op.reshape(_B * _K // 128, 128),
      mp_op.reshape(_B * _K // 128, 128),
  )
  return out[0, 0]


# speedup vs baseline: 14.0368x; 14.0368x over previous
import functools

import jax
import jax.numpy as jnp
from jax import lax
from jax.experimental import pallas as pl
from jax.experimental.pallas import tpu as pltpu
from jax.experimental.pallas import tpu_sc as plsc

_B, _C, _A, _W, _O, _L, _K = 1024, 100, 6, 5, 4, 256, 2
_NC, _NS = 2, 16
_NW = _NC * _NS
_IPW = (_B * _K) // _NW
_NEG = float("-inf")


def _sc_gather_rows(stab, etab, idx):
  mesh = plsc.VectorSubcoreMesh(core_axis_name="c", subcore_axis_name="s")
  f32 = jnp.float32

  @functools.partial(
      pl.kernel,
      mesh=mesh,
      out_type=[
          jax.ShapeDtypeStruct((_B * _K, _L), f32),
          jax.ShapeDtypeStruct((_B * _K, _L), f32),
      ],
      scratch_types=[
          pltpu.VMEM((_IPW,), jnp.int32),
          pltpu.VMEM((_IPW, _L), f32),
          pltpu.VMEM((_IPW, _L), f32),
          pltpu.SemaphoreType.DMA,
          pltpu.SemaphoreType.DMA,
      ],
      compiler_params=pltpu.CompilerParams(needs_layout_passes=False),
  )
  def k(stab_hbm, etab_hbm, idx_hbm, srows_o, erows_o, idx_v, sr_v, er_v,
        sem_s, sem_e):
    wid = lax.axis_index("s") * _NC + lax.axis_index("c")
    i0 = wid * _IPW
    pltpu.sync_copy(idx_hbm.at[pl.ds(i0, _IPW)], idx_v)
    cp_s = pltpu.async_copy(stab_hbm.at[idx_v], sr_v, sem_s)
    cp_e = pltpu.async_copy(etab_hbm.at[idx_v], er_v, sem_e)
    cp_s.wait()
    cp_e.wait()
    pltpu.sync_copy(sr_v, srows_o.at[pl.ds(i0, _IPW)])
    pltpu.sync_copy(er_v, erows_o.at[pl.ds(i0, _IPW)])

  return k(stab, etab, idx)


def _softplus(x):
  return jnp.maximum(x, 0.0) + jnp.log1p(jnp.exp(-jnp.abs(x)))


def _tc_loss_body(sel_ref, num_ref, col_ref, agg_ref, op_ref, srow_ref,
                  erow_ref, selt_ref, numt_ref, aggt_ref, colt_ref, opt_ref,
                  stt_ref, ett_ref, out_ref):
  sel = sel_ref[...]
  ci = lax.broadcasted_iota(jnp.int32, (_C, _B), 0)
  m = jnp.max(sel, axis=0, keepdims=True)
  lse = m + jnp.log(jnp.sum(jnp.exp(sel - m), axis=0, keepdims=True))
  picked = jnp.sum(jnp.where(ci == selt_ref[...], sel, 0.0), axis=0,
                   keepdims=True)
  loss = jnp.sum(lse - picked) * (1.0 / _B)
  amax = jnp.min(jnp.where(sel == m, ci, _C), axis=0, keepdims=True)

  agg = agg_ref[...]
  ci3 = lax.broadcasted_iota(jnp.int32, (_A, _C, _B), 1)
  arow = jnp.sum(jnp.where(ci3 == amax[None], agg, 0.0), axis=1)
  ai = lax.broadcasted_iota(jnp.int32, (_A, _B), 0)
  ma = jnp.max(arow, axis=0, keepdims=True)
  lsea = ma + jnp.log(jnp.sum(jnp.exp(arow - ma), axis=0, keepdims=True))
  pa = jnp.sum(jnp.where(ai == aggt_ref[...], arow, 0.0), axis=0,
               keepdims=True)
  loss = loss + jnp.sum(lsea - pa) * (1.0 / _B)

  num = num_ref[...]
  wi = lax.broadcasted_iota(jnp.int32, (_W, _B), 0)
  mn = jnp.max(num, axis=0, keepdims=True)
  lsen = mn + jnp.log(jnp.sum(jnp.exp(num - mn), axis=0, keepdims=True))
  pn = jnp.sum(jnp.where(wi == numt_ref[...], num, 0.0), axis=0,
               keepdims=True)
  loss = loss + jnp.sum(lsen - pn) * (1.0 / _B)

  colw = col_ref[...]
  t0 = colt_ref[0:1, :]
  t1 = colt_ref[1:2, :]
  h = (ci == t0) | (ci == t1)
  sp_pos = _softplus(colw)
  sp_neg = sp_pos - colw
  loss = loss + jnp.sum(jnp.where(h, 3.0 * sp_neg, sp_pos)) * (
      jnp.float32(_B) / _C)

  op = op_ref[...]
  ci3o = lax.broadcasted_iota(jnp.int32, (_C, _O, _B), 0)
  oi = lax.broadcasted_iota(jnp.int32, (_O, _B), 0)
  for kk in range(_K):
    ck = colt_ref[kk:kk + 1, :]
    orow = jnp.sum(jnp.where(ci3o == ck[:, None], op, 0.0), axis=0)
    mo = jnp.max(orow, axis=0, keepdims=True)
    lseo = mo + jnp.log(jnp.sum(jnp.exp(orow - mo), axis=0, keepdims=True))
    po = jnp.sum(jnp.where(oi == opt_ref[kk:kk + 1, :], orow, 0.0), axis=0,
                 keepdims=True)
    loss = loss + jnp.sum(lseo - po) * (1.0 / (_B * _K))

  jl = lax.broadcasted_iota(jnp.int32, (_B * _K, _L), 1)
  for rows_ref, tgt_ref in ((srow_ref, stt_ref), (erow_ref, ett_ref)):
    x = rows_ref[...]
    mr = jnp.max(x, axis=1, keepdims=True)
    lser = mr + jnp.log(jnp.sum(jnp.exp(x - mr), axis=1, keepdims=True))
    pr = jnp.sum(jnp.where(jl == tgt_ref[...], x, 0.0), axis=1,
                 keepdims=True)
    loss = loss + jnp.sum(lser - pr) * (1.0 / (_B * _K))

  out_ref[...] = jnp.reshape(loss, (1, 1))


def kernel(agg_logits, sel_logits, where_num_logits, where_col_logits,
           where_op_logits, where_start_logits, where_end_logits,
           agg_target, sel_target, where_num_target, where_col_target,
           where_op_target, where_start_target, where_end_target):
  i32 = jnp.int32
  colt_t = where_col_target.astype(i32).T
  idx = (colt_t * _B + jnp.arange(_B, dtype=i32)[None, :]).reshape(-1)

  srows, erows = _sc_gather_rows(
      where_start_logits.transpose(1, 0, 2).reshape(_C * _B, _L),
      where_end_logits.transpose(1, 0, 2).reshape(_C * _B, _L),
      idx)

  out = pl.pallas_call(
      _tc_loss_body,
      out_shape=jax.ShapeDtypeStruct((1, 1), jnp.float32),
  )(
      sel_logits.T,
      where_num_logits.T,
      where_col_logits.T,
      agg_logits.transpose(2, 1, 0),
      where_op_logits.transpose(1, 2, 0),
      srows,
      erows,
      sel_target.astype(i32).reshape(1, _B),
      where_num_target.astype(i32).reshape(1, _B),
      agg_target.astype(i32).reshape(1, _B),
      colt_t,
      where_op_target.astype(i32).T,
      where_start_target.astype(i32).T.reshape(_B * _K, 1),
      where_end_target.astype(i32).T.reshape(_B * _K, 1),
  )
  return out[0, 0]
